# baseline (speedup 1.0000x reference)
import functools

import jax
import jax.numpy as jnp
from jax import lax
from jax.experimental import pallas as pl
from jax.experimental.pallas import tpu as pltpu
from jax.experimental.pallas import tpu_sc as plsc

N_LVL = [1563, 6250, 25000, 100000]

_EB = 128
_G_CAP = 8
_NTILES = 16
_EPAD = _EB * _NTILES


def _round_up(x, m):
    return (x + m - 1) // m * m


def _make_sc_agg(n, n_pad, k2, e_pad, compute_deg):
    blocks_per_tile = e_pad // (_EB * _NTILES)
    rows_per_tile = n_pad // _NTILES
    zr = 128
    n_zdma = rows_per_tile // zr
    g = min(_G_CAP, 256 // k2)
    n_sb = blocks_per_tile // g
    n_rem = blocks_per_tile % g
    mesh = plsc.VectorSubcoreMesh(core_axis_name="c", subcore_axis_name="s")

    out_type = [jax.ShapeDtypeStruct((2, n_pad, k2), jnp.float32)]
    if compute_deg:
        out_type.append(jax.ShapeDtypeStruct((2, n_pad), jnp.float32))
    scratch = [
        pltpu.VMEM((g, _EB), jnp.int32),
        pltpu.VMEM((g, _EB), jnp.int32),
        pltpu.VMEM((g * _EB, k2), jnp.float32),
        pltpu.VMEM((zr, k2), jnp.float32),
        pltpu.SemaphoreType.DMA,
        pltpu.SemaphoreType.DMA,
        pltpu.VMEM_SHARED((n_pad, k2), jnp.float32),
    ]
    if compute_deg:
        scratch += [
            pltpu.VMEM((_EB,), jnp.float32),
            pltpu.VMEM((zr * k2,), jnp.float32),
            pltpu.VMEM_SHARED((n_pad,), jnp.float32),
        ]

    def body(h3, src, dst, *outs_scratch):
        if compute_deg:
            (agg_out, deg_out, srcv, dstv, rows, stage, sem, sems, acc,
             onesv, dstage, dacc) = outs_scratch
        else:
            agg_out, srcv, dstv, rows, stage, sem, sems, acc = outs_scratch
        cid = lax.axis_index("c")
        sid = lax.axis_index("s")
        r0 = sid * rows_per_tile

        def zstage(i, _):
            stage[i // (k2 // 16), pl.ds((i % (k2 // 16)) * 16, 16)] = (
                jnp.zeros((16,), jnp.float32))
            return _
        lax.fori_loop(0, zr * (k2 // 16), zstage, None)

        def zdma(k, _):
            pltpu.sync_copy(stage, acc.at[pl.ds(r0 + k * zr, zr), :])
            return _
        lax.fori_loop(0, n_zdma, zdma, None)

        if compute_deg:
            def zdeg(i, _):
                dstage[pl.ds(i * 16, 16)] = jnp.zeros((16,), jnp.float32)
                return _
            lax.fori_loop(0, zr * k2 // 16, zdeg, None)
            dz = zr * k2
            pos = 0
            while pos < rows_per_tile:
                c = min(dz, rows_per_tile - pos)
                pltpu.sync_copy(dstage.at[pl.ds(0, c)],
                                dacc.at[pl.ds(r0 + pos, c)])
                pos += c
            def onesf(i, _):
                onesv[pl.ds(i * 16, 16)] = jnp.full((16,), 0.5, jnp.float32)
                return _
            lax.fori_loop(0, _EB // 16, onesf, None)

        plsc.subcore_barrier()

        hview = h3.at[cid]
        row0 = sid * blocks_per_tile

        def do_superblock(base_blk, gg):
            base_e = base_blk * _EB
            descs = [
                pltpu.async_copy(src.at[pl.ds(base_e + q * _EB, _EB)],
                                 srcv.at[q], sem)
                for q in range(gg)]
            descs += [
                pltpu.async_copy(dst.at[pl.ds(base_e + q * _EB, _EB)],
                                 dstv.at[q], sem)
                for q in range(gg)]
            for dsc in descs:
                dsc.wait()
            descs = [
                pltpu.async_copy(hview.at[srcv.at[q]],
                                 rows.at[pl.ds(q * _EB, _EB), :], sem)
                for q in range(gg)]
            for dsc in descs:
                dsc.wait()
            descs = []
            for q in range(gg):
                descs.append(pltpu.async_copy(
                    rows.at[pl.ds(q * _EB, _EB), :], acc.at[dstv.at[q]],
                    sems, add=True))
                if compute_deg:
                    descs.append(pltpu.async_copy(
                        onesv, dacc.at[dstv.at[q]], sems, add=True))
            for dsc in descs:
                dsc.wait()

        def sb_body(j, _):
            do_superblock(row0 + j * g, g)
            return _
        lax.fori_loop(0, n_sb, sb_body, None)
        if n_rem:
            do_superblock(row0 + n_sb * g, n_rem)

        plsc.subcore_barrier()

        aview = agg_out.at[cid]

        def wo(k, _):
            pltpu.sync_copy(acc.at[pl.ds(r0 + k * zr, zr), :], stage)
            pltpu.sync_copy(stage, aview.at[pl.ds(r0 + k * zr, zr), :])
            return _
        lax.fori_loop(0, n_zdma, wo, None)

        if compute_deg:
            dview = deg_out.at[cid]
            pos = 0
            while pos < rows_per_tile:
                c = min(zr * k2, rows_per_tile - pos)
                pltpu.sync_copy(dacc.at[pl.ds(r0 + pos, c)],
                                dstage.at[pl.ds(0, c)])
                pltpu.sync_copy(dstage.at[pl.ds(0, c)],
                                dview.at[pl.ds(r0 + pos, c)])
                pos += c

    return pl.kernel(body, out_type=tuple(out_type), mesh=mesh,
                     scratch_types=scratch,
                     compiler_params=pltpu.CompilerParams(
                         use_tc_tiling_on_sc=False))


def _sc_agg_h3(h3, src_pad, dst_pad, n, compute_deg):
    n_pad, k2 = h3.shape[1], h3.shape[2]
    fn = _make_sc_agg(n, n_pad, k2, src_pad.size, compute_deg)
    if compute_deg:
        return fn(h3, src_pad, dst_pad)
    (agg3,) = fn(h3, src_pad, dst_pad)
    return agg3, None


def _make_sc_unpool(nf_pad, k, nc):
    g = max(1, min(_G_CAP, 256 // k))
    n_blocks = nf_pad // _EB
    qb = -(-n_blocks // 32)
    mesh = plsc.VectorSubcoreMesh(core_axis_name="c", subcore_axis_name="s")

    scratch = [
        pltpu.VMEM((g, _EB), jnp.int32),
        pltpu.VMEM((g * _EB, k), jnp.float32),
        pltpu.SemaphoreType.DMA,
    ]

    def body(table, cluster, out, idxv, rows, sem):
        cid = lax.axis_index("c")
        sid = lax.axis_index("s")
        wid = sid * 2 + cid
        b0 = wid * qb
        nb = jnp.clip(n_blocks - b0, 0, qb)

        def do_sb(base_blk, gg):
            descs = [
                pltpu.async_copy(cluster.at[pl.ds((base_blk + q) * _EB, _EB)],
                                 idxv.at[q], sem)
                for q in range(gg)]
            for dsc in descs:
                dsc.wait()
            descs = [
                pltpu.async_copy(table.at[idxv.at[q]],
                                 rows.at[pl.ds(q * _EB, _EB), :], sem)
                for q in range(gg)]
            for dsc in descs:
                dsc.wait()
            descs = [
                pltpu.async_copy(rows.at[pl.ds(q * _EB, _EB), :],
                                 out.at[pl.ds((base_blk + q) * _EB, _EB), :],
                                 sem)
                for q in range(gg)]
            for dsc in descs:
                dsc.wait()

        n_sb = nb // g

        def sb_body(j, _):
            do_sb(b0 + j * g, g)
            return _
        lax.fori_loop(0, n_sb, sb_body, None)

        def rem_body(j, _):
            do_sb(b0 + j, 1)
            return _
        lax.fori_loop(n_sb * g, nb, rem_body, None)

    return pl.kernel(
        body,
        out_type=jax.ShapeDtypeStruct((nf_pad, k), jnp.float32),
        mesh=mesh, scratch_types=scratch,
        compiler_params=pltpu.CompilerParams(use_tc_tiling_on_sc=False))


def _sc_unpool(table_nm, cluster, nf):
    nc, k = table_nm.shape
    nf_pad = _round_up(nf, 2048)
    cl = jnp.concatenate(
        (cluster, jnp.zeros((nf_pad - nf,), jnp.int32)))
    return _make_sc_unpool(nf_pad, k, nc)(table_nm, cl)


def _make_sc_pool(k, nf_pad, nc, nc_pad):
    cpw = k // 32
    cb = 2048
    n_chunks = nf_pad // cb
    acc_n = nc_pad + 2048
    mesh = plsc.VectorSubcoreMesh(core_axis_name="c", subcore_axis_name="s")

    scratch = [
        pltpu.VMEM((cb,), jnp.int32),
        pltpu.VMEM((cpw, cb), jnp.float32),
        pltpu.VMEM((cpw * acc_n,), jnp.float32),
        pltpu.SemaphoreType.DMA,
    ]

    def body(x_cm, cluster, p_out, cluv, xbuf, acc, sem):
        cid = lax.axis_index("c")
        sid = lax.axis_index("s")
        wid = sid * 2 + cid
        ch0 = wid * cpw

        def zf(i, _):
            acc[pl.ds(i * 16, 16)] = jnp.zeros((16,), jnp.float32)
            return _
        lax.fori_loop(0, cpw * acc_n // 16, zf, None)

        def chunk(c, _):
            pos = c * cb
            descs = [pltpu.async_copy(cluster.at[pl.ds(pos, cb)], cluv, sem)]
            descs += [
                pltpu.async_copy(x_cm.at[ch0 + ci].at[pl.ds(pos, cb)],
                                 xbuf.at[ci], sem)
                for ci in range(cpw)]
            for dsc in descs:
                dsc.wait()

            def vec(j, _):
                idx = cluv[pl.ds(j * 16, 16)]
                for ci in range(cpw):
                    idxo = idx + ci * acc_n
                    val = xbuf[ci, pl.ds(j * 16, 16)]
                    cur = plsc.load_gather(acc, [idxo])
                    new = jnp.maximum(cur, val)
                    plsc.store_scatter(acc, [idxo], new)

                    def cond(carry):
                        _, lost = carry
                        return jnp.any(lost)

                    def retry(carry):
                        new, lost = carry
                        cur = plsc.load_gather(acc, [idxo])
                        new = jnp.maximum(cur, new)
                        plsc.store_scatter(acc, [idxo], new, mask=lost)
                        back = plsc.load_gather(acc, [idxo])
                        return new, back < new

                    back = plsc.load_gather(acc, [idxo])
                    lax.while_loop(cond, retry, (new, back < new))
                return _
            lax.fori_loop(0, cb // 16, vec, None)
            return _
        lax.fori_loop(0, n_chunks, chunk, None)

        for ci in range(cpw):
            pltpu.sync_copy(acc.at[pl.ds(ci * acc_n, nc_pad)],
                            p_out.at[ch0 + ci])

    return pl.kernel(
        body,
        out_type=jax.ShapeDtypeStruct((k, nc_pad), jnp.float32),
        mesh=mesh, scratch_types=scratch,
        compiler_params=pltpu.CompilerParams(needs_layout_passes=False))


def _sc_pool(x_cm_pad, cluster, nc):
    k, nf_pad = x_cm_pad.shape
    nc_pad = _round_up(nc, 2048)
    cl = jnp.concatenate(
        (cluster, jnp.full((nf_pad - cluster.shape[0],), nc, jnp.int32)))
    return _make_sc_pool(k, nf_pad, nc, nc_pad)(x_cm_pad, cl)


def _pad_edges(edge_index, n):
    e = edge_index.shape[1]
    ep = _round_up(e, _EPAD)
    src = jnp.concatenate(
        (edge_index[0], jnp.zeros((ep - e,), jnp.int32)))
    dst = jnp.concatenate(
        (edge_index[1], jnp.full((ep - e,), n, jnp.int32)))
    return src, dst


_NB = 512


def _cmm(a, b_t):
    return lax.dot_general(a, b_t, (((1,), (1,)), ((), ())),
                           preferred_element_type=jnp.float32)


def _cmm_cm(x_cm_blk, w):
    return lax.dot_general(x_cm_blk, w, (((0,), (1,)), ((), ())),
                           preferred_element_type=jnp.float32)


def _enc_mm(x_cm, wn, ws, np_):
    c, _ = x_cm.shape
    o, o2 = wn.shape[0], ws.shape[0]

    def body(x_ref, wn_ref, ws_ref, h3_ref, s_ref):
        h = _cmm_cm(x_ref[...], wn_ref[...])
        h3_ref[0] = h[:, :o // 2]
        h3_ref[1] = h[:, o // 2:]
        s_ref[...] = _cmm_cm(x_ref[...], ws_ref[...])

    return pl.pallas_call(
        body, grid=(np_ // _NB,),
        in_specs=[pl.BlockSpec((c, _NB), lambda i: (0, i)),
                  pl.BlockSpec((o, c), lambda i: (0, 0)),
                  pl.BlockSpec((o2, c), lambda i: (0, 0))],
        out_specs=[pl.BlockSpec((2, _NB, o // 2), lambda i: (0, i, 0)),
                   pl.BlockSpec((_NB, o2), lambda i: (i, 0))],
        out_shape=[jax.ShapeDtypeStruct((2, np_, o // 2), jnp.float32),
                   jax.ShapeDtypeStruct((np_, o2), jnp.float32)],
    )(x_cm, wn, ws)


def _s_mm(x_cm, ws, np_):
    c, _ = x_cm.shape
    o = ws.shape[0]

    def body(x_ref, w_ref, s_ref):
        s_ref[...] = _cmm_cm(x_ref[...], w_ref[...])

    return pl.pallas_call(
        body, grid=(np_ // _NB,),
        in_specs=[pl.BlockSpec((c, _NB), lambda i: (0, i)),
                  pl.BlockSpec((o, c), lambda i: (0, 0))],
        out_specs=pl.BlockSpec((_NB, o), lambda i: (i, 0)),
        out_shape=jax.ShapeDtypeStruct((np_, o), jnp.float32),
    )(x_cm, ws)


def _t_h3(x_cm):
    k, np_ = x_cm.shape

    def body(x_ref, h3_ref):
        xt = x_ref[...].T
        h3_ref[0] = xt[:, :k // 2]
        h3_ref[1] = xt[:, k // 2:]

    return pl.pallas_call(
        body, grid=(np_ // _NB,),
        in_specs=[pl.BlockSpec((k, _NB), lambda i: (0, i))],
        out_specs=pl.BlockSpec((2, _NB, k // 2), lambda i: (0, i, 0)),
        out_shape=jax.ShapeDtypeStruct((2, np_, k // 2), jnp.float32),
    )(x_cm)


def _dec_mm(u, skip_cm, wn, ws, c1):
    np_, _ = u.shape
    c2 = skip_cm.shape[0]
    o = wn.shape[0]

    def body(u_ref, sk_ref, wna, wnb, wsa, wsb, h3_ref, s_ref):
        h = _cmm(u_ref[...], wna[...]) + _cmm_cm(sk_ref[...], wnb[...])
        h3_ref[0] = h[:, :o // 2]
        h3_ref[1] = h[:, o // 2:]
        s_ref[...] = (_cmm(u_ref[...], wsa[...])
                      + _cmm_cm(sk_ref[...], wsb[...]))

    return pl.pallas_call(
        body, grid=(np_ // _NB,),
        in_specs=[pl.BlockSpec((_NB, c1), lambda i: (i, 0)),
                  pl.BlockSpec((c2, _NB), lambda i: (0, i)),
                  pl.BlockSpec((o, c1), lambda i: (0, 0)),
                  pl.BlockSpec((o, c2), lambda i: (0, 0)),
                  pl.BlockSpec((o, c1), lambda i: (0, 0)),
                  pl.BlockSpec((o, c2), lambda i: (0, 0))],
        out_specs=[pl.BlockSpec((2, _NB, o // 2), lambda i: (0, i, 0)),
                   pl.BlockSpec((_NB, o), lambda i: (i, 0))],
        out_shape=[jax.ShapeDtypeStruct((2, np_, o // 2), jnp.float32),
                   jax.ShapeDtypeStruct((np_, o), jnp.float32)],
    )(u, skip_cm, wn[:, :c1], wn[:, c1:], ws[:, :c1], ws[:, c1:])


def _finalize(s_nm, agg3, degp, wn, b, cm_out, n_out=None):
    np_ = agg3.shape[1]
    k = agg3.shape[2] * 2
    o = b.shape[0] if wn is None else wn.shape[0]

    def body(*refs):
        if wn is None:
            s_ref, a_ref, d_ref, b_ref, o_ref = refs
        else:
            s_ref, a_ref, d_ref, w_ref, b_ref, o_ref = refs
        agg = jnp.concatenate((a_ref[0], a_ref[1]), axis=1)
        rdeg = 1.0 / jnp.maximum(d_ref[0] + d_ref[1], 1.0)
        t = agg * rdeg[:, None]
        neigh = t if wn is None else _cmm(t, w_ref[...])
        out = jax.nn.relu(s_ref[...] + neigh + b_ref[...])
        o_ref[...] = out.T if cm_out else out

    in_specs = [pl.BlockSpec((_NB, o), lambda i: (i, 0)),
                pl.BlockSpec((2, _NB, k // 2), lambda i: (0, i, 0)),
                pl.BlockSpec((2, _NB), lambda i: (0, i))]
    args = [s_nm, agg3, degp]
    if wn is not None:
        in_specs.append(pl.BlockSpec((o, k), lambda i: (0, 0)))
        args.append(wn)
    in_specs.append(pl.BlockSpec((1, o), lambda i: (0, 0)))
    args.append(b.reshape(1, o))
    if cm_out:
        out_spec = pl.BlockSpec((o, _NB), lambda i: (0, i))
        out_shape = jax.ShapeDtypeStruct((o, n_out or np_), jnp.float32)
    else:
        out_spec = pl.BlockSpec((_NB, o), lambda i: (i, 0))
        out_shape = jax.ShapeDtypeStruct((np_, o), jnp.float32)
    return pl.pallas_call(
        body, grid=(np_ // _NB,), in_specs=in_specs,
        out_specs=out_spec, out_shape=out_shape)(*args)


def kernel(features, enc0_Ws, enc0_Wn, enc0_b, enc1_Ws, enc1_Wn, enc1_b,
           enc2_Ws, enc2_Wn, enc2_b, ubend_Ws, ubend_Wn, ubend_b,
           dec0_Ws, dec0_Wn, dec0_b, dec1_Ws, dec1_Wn, dec1_b,
           dec2_Ws, dec2_Wn, dec2_b, edge_index_0, edge_index_1,
           edge_index_2, edge_index_3, cluster_1, cluster_2, cluster_3):
    n0, n1, n2, n3 = N_LVL
    e3 = _pad_edges(edge_index_3, n3)
    e2 = _pad_edges(edge_index_2, n2)
    e1 = _pad_edges(edge_index_1, n1)
    e0 = _pad_edges(edge_index_0, n0)

    np3, np2, np1, np0 = (_round_up(n, 2048) for n in (n3, n2, n1, n0))

    h3, s = _enc_mm(features, enc0_Wn, enc0_Ws, np3)
    agg3, degp3 = _sc_agg_h3(h3, *e3, n3, True)
    x3e = _finalize(s, agg3, degp3, None, enc0_b, True)

    p3 = _sc_pool(x3e, cluster_3, n2)
    agg3, degp2 = _sc_agg_h3(_t_h3(p3), *e2, n2, True)
    x2e = _finalize(_s_mm(p3, enc1_Ws, np2), agg3, degp2,
                    enc1_Wn, enc1_b, True)

    p2 = _sc_pool(x2e, cluster_2, n1)
    agg3, degp1 = _sc_agg_h3(_t_h3(p2), *e1, n1, True)
    x1e = _finalize(_s_mm(p2, enc2_Ws, np1), agg3, degp1,
                    enc2_Wn, enc2_b, True)

    p1 = _sc_pool(x1e, cluster_1, n0)
    agg3, degp0 = _sc_agg_h3(_t_h3(p1), *e0, n0, True)
    xu_nm = _finalize(_s_mm(p1, ubend_Ws, np0), agg3, degp0,
                      ubend_Wn, ubend_b, False)

    u0 = _sc_unpool(xu_nm, cluster_1, n1)
    h3, s = _dec_mm(u0, x1e, dec0_Wn, dec0_Ws, 256)
    agg3, _ = _sc_agg_h3(h3, *e1, n1, False)
    d1 = _finalize(s, agg3, degp1, None, dec0_b, False)

    u1 = _sc_unpool(d1, cluster_2, n2)
    h3, s = _dec_mm(u1, x2e, dec1_Wn, dec1_Ws, 128)
    agg3, _ = _sc_agg_h3(h3, *e2, n2, False)
    d2 = _finalize(s, agg3, degp2, None, dec1_b, False)

    u2 = _sc_unpool(d2, cluster_3, n3)
    h3, s = _dec_mm(u2, x3e, dec2_Wn, dec2_Ws, 64)
    agg3, _ = _sc_agg_h3(h3, *e3, n3, False)
    return _finalize(s, agg3, degp3, None, dec2_b, True,
                     n_out=n3)


# speedup vs baseline: 1.0437x; 1.0437x over previous
import functools

import jax
import jax.numpy as jnp
from jax import lax
from jax.experimental import pallas as pl
from jax.experimental.pallas import tpu as pltpu
from jax.experimental.pallas import tpu_sc as plsc

N_LVL = [1563, 6250, 25000, 100000]

_EB = 128
_G_CAP = 8
_NTILES = 16
_EPAD = _EB * _NTILES


def _round_up(x, m):
    return (x + m - 1) // m * m


def _make_sc_agg(n, n_pad, k2, e_pad, compute_deg):
    blocks_per_tile = e_pad // (_EB * _NTILES)
    rows_per_tile = n_pad // _NTILES
    zr = 128
    n_zdma = rows_per_tile // zr
    g = min(_G_CAP, 256 // k2)
    n_sb = blocks_per_tile // g
    n_rem = blocks_per_tile % g
    mesh = plsc.VectorSubcoreMesh(core_axis_name="c", subcore_axis_name="s")

    out_type = [jax.ShapeDtypeStruct((2, n_pad, k2), jnp.float32)]
    if compute_deg:
        out_type.append(jax.ShapeDtypeStruct((2, n_pad), jnp.float32))
    scratch = [
        pltpu.VMEM((g, _EB), jnp.int32),
        pltpu.VMEM((g, _EB), jnp.int32),
        pltpu.VMEM((g * _EB, k2), jnp.float32),
        pltpu.VMEM((zr, k2), jnp.float32),
        pltpu.SemaphoreType.DMA,
        pltpu.SemaphoreType.DMA,
        pltpu.VMEM_SHARED((n_pad, k2), jnp.float32),
    ]
    if compute_deg:
        scratch += [
            pltpu.VMEM((_EB,), jnp.float32),
            pltpu.VMEM((zr * k2,), jnp.float32),
            pltpu.VMEM_SHARED((n_pad,), jnp.float32),
        ]

    def body(h3, src, dst, *outs_scratch):
        if compute_deg:
            (agg_out, deg_out, srcv, dstv, rows, stage, sem, sems, acc,
             onesv, dstage, dacc) = outs_scratch
        else:
            agg_out, srcv, dstv, rows, stage, sem, sems, acc = outs_scratch
        cid = lax.axis_index("c")
        sid = lax.axis_index("s")
        r0 = sid * rows_per_tile

        def zstage(i, _):
            stage[i // (k2 // 16), pl.ds((i % (k2 // 16)) * 16, 16)] = (
                jnp.zeros((16,), jnp.float32))
            return _
        lax.fori_loop(0, zr * (k2 // 16), zstage, None)

        def zdma(k, _):
            pltpu.sync_copy(stage, acc.at[pl.ds(r0 + k * zr, zr), :])
            return _
        lax.fori_loop(0, n_zdma, zdma, None)

        if compute_deg:
            def zdeg(i, _):
                dstage[pl.ds(i * 16, 16)] = jnp.zeros((16,), jnp.float32)
                return _
            lax.fori_loop(0, zr * k2 // 16, zdeg, None)
            dz = zr * k2
            pos = 0
            while pos < rows_per_tile:
                c = min(dz, rows_per_tile - pos)
                pltpu.sync_copy(dstage.at[pl.ds(0, c)],
                                dacc.at[pl.ds(r0 + pos, c)])
                pos += c
            def onesf(i, _):
                onesv[pl.ds(i * 16, 16)] = jnp.full((16,), 0.5, jnp.float32)
                return _
            lax.fori_loop(0, _EB // 16, onesf, None)

        plsc.subcore_barrier()

        hview = h3.at[cid]
        row0 = sid * blocks_per_tile

        def do_superblock(base_blk, gg):
            base_e = base_blk * _EB
            descs = [
                pltpu.async_copy(src.at[pl.ds(base_e + q * _EB, _EB)],
                                 srcv.at[q], sem)
                for q in range(gg)]
            descs += [
                pltpu.async_copy(dst.at[pl.ds(base_e + q * _EB, _EB)],
                                 dstv.at[q], sem)
                for q in range(gg)]
            for dsc in descs:
                dsc.wait()
            gdescs = [
                pltpu.async_copy(hview.at[srcv.at[q]],
                                 rows.at[pl.ds(q * _EB, _EB), :], sem)
                for q in range(gg)]
            sdescs = []
            for q in range(gg):
                gdescs[q].wait()
                sdescs.append(pltpu.async_copy(
                    rows.at[pl.ds(q * _EB, _EB), :], acc.at[dstv.at[q]],
                    sems, add=True))
                if compute_deg:
                    sdescs.append(pltpu.async_copy(
                        onesv, dacc.at[dstv.at[q]], sems, add=True))
            for dsc in sdescs:
                dsc.wait()

        def sb_body(j, _):
            do_superblock(row0 + j * g, g)
            return _
        lax.fori_loop(0, n_sb, sb_body, None)
        if n_rem:
            do_superblock(row0 + n_sb * g, n_rem)

        plsc.subcore_barrier()

        aview = agg_out.at[cid]

        def wo(k, _):
            pltpu.sync_copy(acc.at[pl.ds(r0 + k * zr, zr), :], stage)
            pltpu.sync_copy(stage, aview.at[pl.ds(r0 + k * zr, zr), :])
            return _
        lax.fori_loop(0, n_zdma, wo, None)

        if compute_deg:
            dview = deg_out.at[cid]
            pos = 0
            while pos < rows_per_tile:
                c = min(zr * k2, rows_per_tile - pos)
                pltpu.sync_copy(dacc.at[pl.ds(r0 + pos, c)],
                                dstage.at[pl.ds(0, c)])
                pltpu.sync_copy(dstage.at[pl.ds(0, c)],
                                dview.at[pl.ds(r0 + pos, c)])
                pos += c

    return pl.kernel(body, out_type=tuple(out_type), mesh=mesh,
                     scratch_types=scratch,
                     compiler_params=pltpu.CompilerParams(
                         use_tc_tiling_on_sc=False))


def _sc_agg_h3(h3, src_pad, dst_pad, n, compute_deg):
    n_pad, k2 = h3.shape[1], h3.shape[2]
    fn = _make_sc_agg(n, n_pad, k2, src_pad.size, compute_deg)
    if compute_deg:
        return fn(h3, src_pad, dst_pad)
    (agg3,) = fn(h3, src_pad, dst_pad)
    return agg3, None


def _make_sc_unpool(nf_pad, k, nc):
    g = max(1, min(_G_CAP, 256 // k))
    n_blocks = nf_pad // _EB
    qb = -(-n_blocks // 32)
    mesh = plsc.VectorSubcoreMesh(core_axis_name="c", subcore_axis_name="s")

    scratch = [
        pltpu.VMEM((g, _EB), jnp.int32),
        pltpu.VMEM((g * _EB, k), jnp.float32),
        pltpu.SemaphoreType.DMA,
    ]

    def body(table, cluster, out, idxv, rows, sem):
        cid = lax.axis_index("c")
        sid = lax.axis_index("s")
        wid = sid * 2 + cid
        b0 = wid * qb
        nb = jnp.clip(n_blocks - b0, 0, qb)

        def do_sb(base_blk, gg):
            descs = [
                pltpu.async_copy(cluster.at[pl.ds((base_blk + q) * _EB, _EB)],
                                 idxv.at[q], sem)
                for q in range(gg)]
            for dsc in descs:
                dsc.wait()
            descs = [
                pltpu.async_copy(table.at[idxv.at[q]],
                                 rows.at[pl.ds(q * _EB, _EB), :], sem)
                for q in range(gg)]
            for dsc in descs:
                dsc.wait()
            descs = [
                pltpu.async_copy(rows.at[pl.ds(q * _EB, _EB), :],
                                 out.at[pl.ds((base_blk + q) * _EB, _EB), :],
                                 sem)
                for q in range(gg)]
            for dsc in descs:
                dsc.wait()

        n_sb = nb // g

        def sb_body(j, _):
            do_sb(b0 + j * g, g)
            return _
        lax.fori_loop(0, n_sb, sb_body, None)

        def rem_body(j, _):
            do_sb(b0 + j, 1)
            return _
        lax.fori_loop(n_sb * g, nb, rem_body, None)

    return pl.kernel(
        body,
        out_type=jax.ShapeDtypeStruct((nf_pad, k), jnp.float32),
        mesh=mesh, scratch_types=scratch,
        compiler_params=pltpu.CompilerParams(use_tc_tiling_on_sc=False))


def _sc_unpool(table_nm, cluster, nf):
    nc, k = table_nm.shape
    nf_pad = _round_up(nf, 2048)
    cl = jnp.concatenate(
        (cluster, jnp.zeros((nf_pad - nf,), jnp.int32)))
    return _make_sc_unpool(nf_pad, k, nc)(table_nm, cl)


def _make_sc_pool(k, nf_pad, nc, nc_pad):
    cpw = k // 32
    cb = 2048
    n_chunks = nf_pad // cb
    acc_n = nc_pad + 2048
    mesh = plsc.VectorSubcoreMesh(core_axis_name="c", subcore_axis_name="s")

    scratch = [
        pltpu.VMEM((cb,), jnp.int32),
        pltpu.VMEM((cpw, cb), jnp.float32),
        pltpu.VMEM((cpw * acc_n,), jnp.float32),
        pltpu.SemaphoreType.DMA,
    ]

    def body(x_cm, cluster, p_out, cluv, xbuf, acc, sem):
        cid = lax.axis_index("c")
        sid = lax.axis_index("s")
        wid = sid * 2 + cid
        ch0 = wid * cpw

        def zf(i, _):
            acc[pl.ds(i * 16, 16)] = jnp.zeros((16,), jnp.float32)
            return _
        lax.fori_loop(0, cpw * acc_n // 16, zf, None)

        def chunk(c, _):
            pos = c * cb
            descs = [pltpu.async_copy(cluster.at[pl.ds(pos, cb)], cluv, sem)]
            descs += [
                pltpu.async_copy(x_cm.at[ch0 + ci].at[pl.ds(pos, cb)],
                                 xbuf.at[ci], sem)
                for ci in range(cpw)]
            for dsc in descs:
                dsc.wait()

            def vec(j, carry):
                lanes = []
                for u in range(2):
                    idx = cluv[pl.ds((2 * j + u) * 16, 16)]
                    for ci in range(cpw):
                        lanes.append(
                            (idx + ci * acc_n,
                             xbuf[ci, pl.ds((2 * j + u) * 16, 16)]))
                news = []
                for idxo, val in lanes:
                    cur = plsc.load_gather(acc, [idxo])
                    news.append(jnp.maximum(cur, val))
                for (idxo, _), new in zip(lanes, news):
                    plsc.store_scatter(acc, [idxo], new)
                for (idxo, _), new in zip(lanes, news):
                    def cond(carry):
                        _, lost = carry
                        return jnp.any(lost)

                    def retry(carry, idxo=idxo):
                        new, lost = carry
                        cur = plsc.load_gather(acc, [idxo])
                        new = jnp.maximum(cur, new)
                        plsc.store_scatter(acc, [idxo], new, mask=lost)
                        back = plsc.load_gather(acc, [idxo])
                        return new, back < new

                    back = plsc.load_gather(acc, [idxo])
                    lax.while_loop(cond, retry, (new, back < new))
                return carry
            lax.fori_loop(0, cb // 32, vec, None)
            return _
        lax.fori_loop(0, n_chunks, chunk, None)

        for ci in range(cpw):
            pltpu.sync_copy(acc.at[pl.ds(ci * acc_n, nc_pad)],
                            p_out.at[ch0 + ci])

    return pl.kernel(
        body,
        out_type=jax.ShapeDtypeStruct((k, nc_pad), jnp.float32),
        mesh=mesh, scratch_types=scratch,
        compiler_params=pltpu.CompilerParams(needs_layout_passes=False))


def _sc_pool(x_cm_pad, cluster, nc):
    k, nf_pad = x_cm_pad.shape
    nc_pad = _round_up(nc, 2048)
    cl = jnp.concatenate(
        (cluster, jnp.full((nf_pad - cluster.shape[0],), nc, jnp.int32)))
    return _make_sc_pool(k, nf_pad, nc, nc_pad)(x_cm_pad, cl)


def _pad_edges(edge_index, n):
    e = edge_index.shape[1]
    ep = _round_up(e, _EPAD)
    src = jnp.concatenate(
        (edge_index[0], jnp.zeros((ep - e,), jnp.int32)))
    dst = jnp.concatenate(
        (edge_index[1], jnp.full((ep - e,), n, jnp.int32)))
    return src, dst


_NB = 512


def _cmm(a, b_t):
    return lax.dot_general(a, b_t, (((1,), (1,)), ((), ())),
                           preferred_element_type=jnp.float32)


def _cmm_cm(x_cm_blk, w):
    return lax.dot_general(x_cm_blk, w, (((0,), (1,)), ((), ())),
                           preferred_element_type=jnp.float32)


def _enc_mm(x_cm, wn, ws, np_):
    c, _ = x_cm.shape
    o, o2 = wn.shape[0], ws.shape[0]

    def body(x_ref, wn_ref, ws_ref, h3_ref, s_ref):
        h = _cmm_cm(x_ref[...], wn_ref[...])
        h3_ref[0] = h[:, :o // 2]
        h3_ref[1] = h[:, o // 2:]
        s_ref[...] = _cmm_cm(x_ref[...], ws_ref[...])

    return pl.pallas_call(
        body, grid=(np_ // _NB,),
        in_specs=[pl.BlockSpec((c, _NB), lambda i: (0, i)),
                  pl.BlockSpec((o, c), lambda i: (0, 0)),
                  pl.BlockSpec((o2, c), lambda i: (0, 0))],
        out_specs=[pl.BlockSpec((2, _NB, o // 2), lambda i: (0, i, 0)),
                   pl.BlockSpec((_NB, o2), lambda i: (i, 0))],
        out_shape=[jax.ShapeDtypeStruct((2, np_, o // 2), jnp.float32),
                   jax.ShapeDtypeStruct((np_, o2), jnp.float32)],
    )(x_cm, wn, ws)


def _s_mm(x_cm, ws, np_):
    c, _ = x_cm.shape
    o = ws.shape[0]

    def body(x_ref, w_ref, s_ref):
        s_ref[...] = _cmm_cm(x_ref[...], w_ref[...])

    return pl.pallas_call(
        body, grid=(np_ // _NB,),
        in_specs=[pl.BlockSpec((c, _NB), lambda i: (0, i)),
                  pl.BlockSpec((o, c), lambda i: (0, 0))],
        out_specs=pl.BlockSpec((_NB, o), lambda i: (i, 0)),
        out_shape=jax.ShapeDtypeStruct((np_, o), jnp.float32),
    )(x_cm, ws)


def _t_h3(x_cm):
    k, np_ = x_cm.shape

    def body(x_ref, h3_ref):
        xt = x_ref[...].T
        h3_ref[0] = xt[:, :k // 2]
        h3_ref[1] = xt[:, k // 2:]

    return pl.pallas_call(
        body, grid=(np_ // _NB,),
        in_specs=[pl.BlockSpec((k, _NB), lambda i: (0, i))],
        out_specs=pl.BlockSpec((2, _NB, k // 2), lambda i: (0, i, 0)),
        out_shape=jax.ShapeDtypeStruct((2, np_, k // 2), jnp.float32),
    )(x_cm)


def _dec_mm(u, skip_cm, wn, ws, c1):
    np_, _ = u.shape
    c2 = skip_cm.shape[0]
    o = wn.shape[0]

    def body(u_ref, sk_ref, wna, wnb, wsa, wsb, h3_ref, s_ref):
        h = _cmm(u_ref[...], wna[...]) + _cmm_cm(sk_ref[...], wnb[...])
        h3_ref[0] = h[:, :o // 2]
        h3_ref[1] = h[:, o // 2:]
        s_ref[...] = (_cmm(u_ref[...], wsa[...])
                      + _cmm_cm(sk_ref[...], wsb[...]))

    return pl.pallas_call(
        body, grid=(np_ // _NB,),
        in_specs=[pl.BlockSpec((_NB, c1), lambda i: (i, 0)),
                  pl.BlockSpec((c2, _NB), lambda i: (0, i)),
                  pl.BlockSpec((o, c1), lambda i: (0, 0)),
                  pl.BlockSpec((o, c2), lambda i: (0, 0)),
                  pl.BlockSpec((o, c1), lambda i: (0, 0)),
                  pl.BlockSpec((o, c2), lambda i: (0, 0))],
        out_specs=[pl.BlockSpec((2, _NB, o // 2), lambda i: (0, i, 0)),
                   pl.BlockSpec((_NB, o), lambda i: (i, 0))],
        out_shape=[jax.ShapeDtypeStruct((2, np_, o // 2), jnp.float32),
                   jax.ShapeDtypeStruct((np_, o), jnp.float32)],
    )(u, skip_cm, wn[:, :c1], wn[:, c1:], ws[:, :c1], ws[:, c1:])


def _finalize(s_nm, agg3, degp, wn, b, cm_out, n_out=None):
    np_ = agg3.shape[1]
    k = agg3.shape[2] * 2
    o = b.shape[0] if wn is None else wn.shape[0]

    def body(*refs):
        if wn is None:
            s_ref, a_ref, d_ref, b_ref, o_ref = refs
        else:
            s_ref, a_ref, d_ref, w_ref, b_ref, o_ref = refs
        agg = jnp.concatenate((a_ref[0], a_ref[1]), axis=1)
        rdeg = 1.0 / jnp.maximum(d_ref[0] + d_ref[1], 1.0)
        t = agg * rdeg[:, None]
        neigh = t if wn is None else _cmm(t, w_ref[...])
        out = jax.nn.relu(s_ref[...] + neigh + b_ref[...])
        o_ref[...] = out.T if cm_out else out

    in_specs = [pl.BlockSpec((_NB, o), lambda i: (i, 0)),
                pl.BlockSpec((2, _NB, k // 2), lambda i: (0, i, 0)),
                pl.BlockSpec((2, _NB), lambda i: (0, i))]
    args = [s_nm, agg3, degp]
    if wn is not None:
        in_specs.append(pl.BlockSpec((o, k), lambda i: (0, 0)))
        args.append(wn)
    in_specs.append(pl.BlockSpec((1, o), lambda i: (0, 0)))
    args.append(b.reshape(1, o))
    if cm_out:
        out_spec = pl.BlockSpec((o, _NB), lambda i: (0, i))
        out_shape = jax.ShapeDtypeStruct((o, n_out or np_), jnp.float32)
    else:
        out_spec = pl.BlockSpec((_NB, o), lambda i: (i, 0))
        out_shape = jax.ShapeDtypeStruct((np_, o), jnp.float32)
    return pl.pallas_call(
        body, grid=(np_ // _NB,), in_specs=in_specs,
        out_specs=out_spec, out_shape=out_shape)(*args)


def kernel(features, enc0_Ws, enc0_Wn, enc0_b, enc1_Ws, enc1_Wn, enc1_b,
           enc2_Ws, enc2_Wn, enc2_b, ubend_Ws, ubend_Wn, ubend_b,
           dec0_Ws, dec0_Wn, dec0_b, dec1_Ws, dec1_Wn, dec1_b,
           dec2_Ws, dec2_Wn, dec2_b, edge_index_0, edge_index_1,
           edge_index_2, edge_index_3, cluster_1, cluster_2, cluster_3):
    n0, n1, n2, n3 = N_LVL
    e3 = _pad_edges(edge_index_3, n3)
    e2 = _pad_edges(edge_index_2, n2)
    e1 = _pad_edges(edge_index_1, n1)
    e0 = _pad_edges(edge_index_0, n0)

    np3, np2, np1, np0 = (_round_up(n, 2048) for n in (n3, n2, n1, n0))

    h3, s = _enc_mm(features, enc0_Wn, enc0_Ws, np3)
    agg3, degp3 = _sc_agg_h3(h3, *e3, n3, True)
    x3e = _finalize(s, agg3, degp3, None, enc0_b, True)

    p3 = _sc_pool(x3e, cluster_3, n2)
    agg3, degp2 = _sc_agg_h3(_t_h3(p3), *e2, n2, True)
    x2e = _finalize(_s_mm(p3, enc1_Ws, np2), agg3, degp2,
                    enc1_Wn, enc1_b, True)

    p2 = _sc_pool(x2e, cluster_2, n1)
    agg3, degp1 = _sc_agg_h3(_t_h3(p2), *e1, n1, True)
    x1e = _finalize(_s_mm(p2, enc2_Ws, np1), agg3, degp1,
                    enc2_Wn, enc2_b, True)

    p1 = _sc_pool(x1e, cluster_1, n0)
    agg3, degp0 = _sc_agg_h3(_t_h3(p1), *e0, n0, True)
    xu_nm = _finalize(_s_mm(p1, ubend_Ws, np0), agg3, degp0,
                      ubend_Wn, ubend_b, False)

    u0 = _sc_unpool(xu_nm, cluster_1, n1)
    h3, s = _dec_mm(u0, x1e, dec0_Wn, dec0_Ws, 256)
    agg3, _ = _sc_agg_h3(h3, *e1, n1, False)
    d1 = _finalize(s, agg3, degp1, None, dec0_b, False)

    u1 = _sc_unpool(d1, cluster_2, n2)
    h3, s = _dec_mm(u1, x2e, dec1_Wn, dec1_Ws, 128)
    agg3, _ = _sc_agg_h3(h3, *e2, n2, False)
    d2 = _finalize(s, agg3, degp2, None, dec1_b, False)

    u2 = _sc_unpool(d2, cluster_3, n3)
    h3, s = _dec_mm(u2, x3e, dec2_Wn, dec2_Ws, 64)
    agg3, _ = _sc_agg_h3(h3, *e3, n3, False)
    return _finalize(s, agg3, degp3, None, dec2_b, True,
                     n_out=n3)
